# baseline (speedup 1.0000x reference)
import jax
import jax.numpy as jnp
from jax.experimental import pallas as pl
from jax.experimental.pallas import tpu as pltpu

_DD = 8


def _tc_body(x_ref, t_ref, o_ref):
    C = x_ref.shape[1]
    W = x_ref.shape[4]
    x = x_ref[0]
    tgt = t_ref[0]
    best = x[0]
    pred = jnp.zeros_like(tgt)
    for c in range(1, C):
        m = x[c] > best
        best = jnp.where(m, x[c], best)
        pred = jnp.where(m, c, pred)
    one = jnp.ones_like(best)
    zero = jnp.zeros_like(best)
    rows = []
    for c in range(C):
        pc = pred == c
        tc = tgt == c
        for msk in (pc & tc, pc, tc):
            r = jnp.sum(jnp.where(msk, one, zero), axis=(0, 1), keepdims=True)
            rows.append(r.reshape(1, W))
    cnt = jnp.concatenate(rows, axis=0)
    i = pl.program_id(1)

    @pl.when(i == 0)
    def _init():
        o_ref[0] = cnt

    @pl.when(i > 0)
    def _acc():
        o_ref[0] = o_ref[0] + cnt


def kernel(inputs, targets):
    eps = 1e-05
    B, C, D, H, W = inputs.shape
    t = targets.astype(jnp.int32)
    G = D // _DD
    counts = pl.pallas_call(
        _tc_body,
        grid=(B, G),
        in_specs=[
            pl.BlockSpec((1, C, _DD, H, W), lambda b, i: (b, 0, i, 0, 0)),
            pl.BlockSpec((1, _DD, H, W), lambda b, i: (b, i, 0, 0)),
        ],
        out_specs=pl.BlockSpec((1, 3 * C, W), lambda b, i: (b, 0, 0)),
        out_shape=jax.ShapeDtypeStruct((B, 3 * C, W), jnp.float32),
        compiler_params=pltpu.CompilerParams(
            dimension_semantics=("parallel", "arbitrary")),
    )(inputs, t)
    cnt = counts.sum(axis=2).reshape(B, C, 3)
    tp, cp, ct = cnt[..., 0], cnt[..., 1], cnt[..., 2]
    loss = 2.0 * tp / (cp + ct + eps)
    return loss[:, 1:].mean(axis=1)


# speedup vs baseline: 1.0320x; 1.0320x over previous
import jax
import jax.numpy as jnp
from jax.experimental import pallas as pl
from jax.experimental.pallas import tpu as pltpu

_DD = 2


def _tc_body(x_ref, t_ref, o_ref):
    C = x_ref.shape[1]
    W = x_ref.shape[4]
    x = x_ref[0]
    tgt = t_ref[0]
    best = x[0]
    pred = jnp.zeros_like(tgt)
    for c in range(1, C):
        m = x[c] > best
        best = jnp.where(m, x[c], best)
        pred = jnp.where(m, c, pred)
    one = jnp.ones_like(best)
    zero = jnp.zeros_like(best)
    rows = []
    for c in range(C):
        pc = pred == c
        tc = tgt == c
        for msk in (pc & tc, pc, tc):
            r = jnp.sum(jnp.where(msk, one, zero), axis=(0, 1), keepdims=True)
            rows.append(r.reshape(1, W))
    cnt = jnp.concatenate(rows, axis=0)
    i = pl.program_id(1)

    @pl.when(i == 0)
    def _init():
        o_ref[0] = cnt

    @pl.when(i > 0)
    def _acc():
        o_ref[0] = o_ref[0] + cnt


def kernel(inputs, targets):
    eps = 1e-05
    B, C, D, H, W = inputs.shape
    t = targets.astype(jnp.int32)
    G = D // _DD
    counts = pl.pallas_call(
        _tc_body,
        grid=(B, G),
        in_specs=[
            pl.BlockSpec((1, C, _DD, H, W), lambda b, i: (b, 0, i, 0, 0)),
            pl.BlockSpec((1, _DD, H, W), lambda b, i: (b, i, 0, 0)),
        ],
        out_specs=pl.BlockSpec((1, 3 * C, W), lambda b, i: (b, 0, 0)),
        out_shape=jax.ShapeDtypeStruct((B, 3 * C, W), jnp.float32),
        compiler_params=pltpu.CompilerParams(
            dimension_semantics=("parallel", "arbitrary")),
    )(inputs, t)
    cnt = counts.sum(axis=2).reshape(B, C, 3)
    tp, cp, ct = cnt[..., 0], cnt[..., 1], cnt[..., 2]
    loss = 2.0 * tp / (cp + ct + eps)
    return loss[:, 1:].mean(axis=1)
